# baseline (speedup 1.0000x reference)
import functools

import jax
import jax.numpy as jnp
from jax import lax
from jax.experimental import pallas as pl
from jax.experimental.pallas import tpu as pltpu
from jax.experimental.pallas import tpu_sc as plsc

_EMB_DIM = 64
_LAM = 0.0001
_NC = 2
_NS = 16
_LANES = 16
_NW = _NC * _NS

_CHUNK = 512
_SUB = 128
_NSUB = _CHUNK // _SUB


@functools.lru_cache(maxsize=None)
def _build_sc_kernel(B: int, L: int):
    P = B * L
    assert P % (_NW * _CHUNK) == 0, (B, L)
    assert B % (_NW * _SUB) == 0, (B, L)
    per_w = P // _NW
    n_chunks = per_w // _CHUNK
    users_per_w = B // _NW
    nsub_u = users_per_w // _SUB
    groups = _CHUNK // _LANES

    mesh = plsc.VectorSubcoreMesh(core_axis_name="c", subcore_axis_name="s")

    @functools.partial(
        pl.kernel,
        out_type=(
            jax.ShapeDtypeStruct((P,), jnp.float32),
            jax.ShapeDtypeStruct((_NW, _LANES), jnp.float32),
            jax.ShapeDtypeStruct((_NW, _LANES), jnp.float32),
        ),
        mesh=mesh,
        compiler_params=pltpu.CompilerParams(
            needs_layout_passes=False, use_tc_tiling_on_sc=False),
        scratch_types=[
            pltpu.VMEM((nsub_u, _SUB), jnp.int32),
            pltpu.VMEM((users_per_w, _EMB_DIM), jnp.float32),
            pltpu.VMEM((_NSUB, _SUB), jnp.int32),
            pltpu.VMEM((_CHUNK, _EMB_DIM), jnp.float32),
            pltpu.VMEM((_CHUNK,), jnp.float32),
            pltpu.VMEM((_LANES,), jnp.float32),
            pltpu.SemaphoreType.DMA,
        ],
    )
    def sc_kernel(users_hbm, pois_hbm, uemb_hbm, iemb_hbm,
                  pred_hbm, lu_hbm, li_hbm,
                  uidx_v, urows_v, pidx_v, rows_v, out_v, part_v, sem):
        wid = lax.axis_index("s") * _NC + lax.axis_index("c")

        ubase = wid * users_per_w
        for r in range(nsub_u):
            pltpu.sync_copy(users_hbm.at[pl.ds(ubase + r * _SUB, _SUB)],
                            uidx_v.at[r])
        ucps = [
            pltpu.async_copy(uemb_hbm.at[uidx_v.at[r]],
                             urows_v.at[pl.ds(r * _SUB, _SUB)], sem)
            for r in range(nsub_u)
        ]
        for cp in ucps:
            cp.wait()

        pbase = wid * per_w
        lane = lax.iota(jnp.int32, _LANES)

        def chunk_body(c, carry):
            lu, li = carry
            base = pbase + c * _CHUNK
            for r in range(_NSUB):
                pltpu.sync_copy(pois_hbm.at[pl.ds(base + r * _SUB, _SUB)],
                                pidx_v.at[r])
            cps = [
                pltpu.async_copy(iemb_hbm.at[pidx_v.at[r]],
                                 rows_v.at[pl.ds(r * _SUB, _SUB)], sem)
                for r in range(_NSUB)
            ]
            for cp in cps:
                cp.wait()

            local0 = c * _CHUNK

            def group_body(g, carry2):
                lu2, li2 = carry2
                ids = g * _LANES + lane
                ul = lax.div(local0 + ids, jnp.int32(L))

                def d_body(d, carry3):
                    acc, lu3, li3 = carry3
                    dv = jnp.full((_LANES,), d, jnp.int32)
                    iv = plsc.load_gather(rows_v, [ids, dv])
                    uv = plsc.load_gather(urows_v, [ul, dv])
                    return (acc + uv * iv, lu3 + uv * uv, li3 + iv * iv)

                acc, lu2, li2 = lax.fori_loop(
                    0, _EMB_DIM, d_body,
                    (jnp.zeros((_LANES,), jnp.float32), lu2, li2))
                plsc.store_scatter(out_v, [ids], acc)
                return (lu2, li2)

            lu, li = lax.fori_loop(0, groups, group_body, (lu, li))
            pltpu.sync_copy(out_v, pred_hbm.at[pl.ds(base, _CHUNK)])
            return (lu, li)

        zero = jnp.zeros((_LANES,), jnp.float32)
        lu, li = lax.fori_loop(0, n_chunks, chunk_body, (zero, zero))

        part_v[...] = lu
        pltpu.sync_copy(part_v, lu_hbm.at[wid])
        part_v[...] = li
        pltpu.sync_copy(part_v, li_hbm.at[wid])

    return sc_kernel


def kernel(users, POIs, u_embeds, i_embeds):
    B, L = POIs.shape
    P = B * L
    sc = _build_sc_kernel(B, L)
    pred_flat, lu, li = sc(users.reshape(B), POIs.reshape(P),
                           u_embeds, i_embeds)
    pred = pred_flat.reshape(B, L)
    loss = _LAM * (jnp.sum(lu) + jnp.sum(li))
    return (pred, loss)


# speedup vs baseline: 1.5091x; 1.5091x over previous
import functools

import jax
import jax.numpy as jnp
from jax import lax
from jax.experimental import pallas as pl
from jax.experimental.pallas import tpu as pltpu
from jax.experimental.pallas import tpu_sc as plsc

_EMB_DIM = 64
_LAM = 0.0001
_NC = 2
_NS = 16
_LANES = 16
_NW = _NC * _NS

_CHUNK = 128
_NSTREAM = 1
_DUNROLL = 4

@functools.lru_cache(maxsize=None)
def _build_sc_kernel(B: int, L: int, V: int):
    P = B * L
    assert P % (_NW * _CHUNK) == 0, (B, L)
    assert B % _NW == 0, (B, L)
    per_w = P // _NW
    n_chunks = per_w // _CHUNK
    n_k = n_chunks // 2
    users_per_w = B // _NW
    uflat_per_w = users_per_w * _EMB_DIM
    groups = _CHUNK // _LANES

    mesh = plsc.VectorSubcoreMesh(core_axis_name="c", subcore_axis_name="s")

    @functools.partial(
        pl.kernel,
        out_type=(
            jax.ShapeDtypeStruct((P,), jnp.float32),
            jax.ShapeDtypeStruct((_NW * _LANES,), jnp.float32),
            jax.ShapeDtypeStruct((_NW * _LANES,), jnp.float32),
        ),
        mesh=mesh,
        compiler_params=pltpu.CompilerParams(
            needs_layout_passes=False, use_tc_tiling_on_sc=False),
        scratch_types=[
            pltpu.VMEM((per_w,), jnp.int32),
            pltpu.VMEM((uflat_per_w,), jnp.float32),
            pltpu.VMEM((_NSTREAM, _CHUNK // _NSTREAM), jnp.int32),
            pltpu.VMEM((_NSTREAM, _CHUNK // _NSTREAM), jnp.int32),
            pltpu.VMEM((_CHUNK, _EMB_DIM), jnp.float32),
            pltpu.VMEM((_CHUNK, _EMB_DIM), jnp.float32),
            pltpu.VMEM((_CHUNK,), jnp.float32),
            pltpu.VMEM((_CHUNK,), jnp.float32),
            pltpu.VMEM((_LANES,), jnp.float32),
            pltpu.SemaphoreType.DMA,
            pltpu.SemaphoreType.DMA,
            pltpu.SemaphoreType.DMA,
            pltpu.SemaphoreType.DMA,
        ],
    )
    def sc_kernel(pois_hbm, uflat_hbm, iwide_hbm,
                  pred_hbm, lu_hbm, li_hbm,
                  pois_v, uflat_v, widx0, widx1, rows0, rows1,
                  out0, out1, part_v,
                  sem_g0, sem_g1, sem_o0, sem_o1):
        wid = lax.axis_index("s") * _NC + lax.axis_index("c")
        pbase = wid * per_w
        lane = lax.iota(jnp.int32, _LANES)

        widx = (widx0, widx1)
        rows = (rows0, rows1)
        outs = (out0, out1)
        sem_g = (sem_g0, sem_g1)
        sem_o = (sem_o0, sem_o1)

        pltpu.sync_copy(pois_hbm.at[pl.ds(pbase, per_w)], pois_v)
        pltpu.sync_copy(uflat_hbm.at[pl.ds(wid * uflat_per_w, uflat_per_w)],
                        uflat_v)

        sub = _CHUNK // _NSTREAM

        def issue_gather(c, buf):
            for j in range(groups):
                v = plsc.load_gather(pois_v, [c * _CHUNK + j * _LANES + lane])
                widx[buf][j * _LANES // sub,
                          pl.ds(j * _LANES % sub, _LANES)] = v
            for r in range(_NSTREAM):
                pltpu.async_copy(iwide_hbm.at[widx[buf].at[r]],
                                 rows[buf].at[pl.ds(r * sub, sub)],
                                 sem_g[buf])

        def wait_gather(buf):
            pltpu.make_async_copy(iwide_hbm.at[pl.ds(0, _CHUNK)], rows[buf],
                                  sem_g[buf]).wait()

        def wait_out(buf):
            pltpu.make_async_copy(pred_hbm.at[pl.ds(0, _CHUNK)], outs[buf],
                                  sem_o[buf]).wait()

        issue_gather(0, 0)

        def k_body(k, carry):
            lu, li = carry
            for par in (0, 1):
                c = 2 * k + par
                if par == 0:
                    issue_gather(c + 1, 1)
                else:
                    @pl.when(k < n_k - 1)
                    def _():
                        issue_gather(c + 1, 0)
                wait_gather(par)

                @pl.when(k >= 1)
                def _():
                    wait_out(par)

                def group_body(g, carry2):
                    lu2, li2 = carry2
                    ids = g * _LANES + lane
                    ploc = c * _CHUNK + ids
                    ul = lax.div(ploc, jnp.int32(L))
                    ubase = ul * _EMB_DIM
                    zerov = jnp.zeros((_LANES,), jnp.int32)

                    def d_body(j, carry3):
                        acc, lu3, li3 = carry3
                        for dd in range(_DUNROLL):
                            d = j * _DUNROLL + dd
                            iv = plsc.load_gather(rows[par], [ids, zerov + d])
                            uv = plsc.load_gather(uflat_v, [ubase + d])
                            acc = acc + uv * iv
                            lu3 = lu3 + uv * uv
                            li3 = li3 + iv * iv
                        return (acc, lu3, li3)

                    acc, lu2, li2 = lax.fori_loop(
                        0, _EMB_DIM // _DUNROLL, d_body,
                        (jnp.zeros((_LANES,), jnp.float32), lu2, li2))
                    plsc.store_scatter(outs[par], [ids], acc)
                    return (lu2, li2)

                lu, li = lax.fori_loop(0, groups, group_body, (lu, li))
                pltpu.async_copy(outs[par],
                                 pred_hbm.at[pl.ds(pbase + c * _CHUNK, _CHUNK)],
                                 sem_o[par])
            return (lu, li)

        zero = jnp.zeros((_LANES,), jnp.float32)
        lu, li = lax.fori_loop(0, n_k, k_body, (zero, zero))

        wait_out(0)
        wait_out(1)
        part_v[...] = lu
        pltpu.sync_copy(part_v, lu_hbm.at[pl.ds(wid * _LANES, _LANES)])
        part_v[...] = li
        pltpu.sync_copy(part_v, li_hbm.at[pl.ds(wid * _LANES, _LANES)])

    return sc_kernel


def kernel(users, POIs, u_embeds, i_embeds):
    B, L = POIs.shape
    V = i_embeds.shape[0]
    pois_flat = POIs.reshape(B * L)
    uflat = jnp.take(u_embeds, users.reshape(B), axis=0).reshape(B * _EMB_DIM)
    sc = _build_sc_kernel(B, L, V)
    pred_flat, lu, li = sc(pois_flat, uflat, i_embeds)
    pred = pred_flat.reshape(B, L)
    loss = _LAM * (jnp.sum(lu) + jnp.sum(li))
    return (pred, loss)
